# Initial kernel scaffold; baseline (speedup 1.0000x reference)
import jax
import jax.numpy as jnp
from jax.experimental import pallas as pl


def kernel(input, masks, beziers):
    raise NotImplementedError("write your pallas kernel here")


# speedup vs baseline: 13.2046x; 13.2046x over previous
import functools

import jax
import jax.numpy as jnp
from jax.experimental import pallas as pl
from jax.experimental.pallas import tpu as pltpu

POOLED_H, POOLED_W = 16, 64
SCALE = 0.25


def _prep_body(x_ref, m_ref, o_ref):
    j = pl.program_id(1)
    live = (j < 8).astype(jnp.float32)
    o_ref[0] = x_ref[0] * m_ref[...] * live


def _main_body(xm_ref, px_ref, py_ref, o_ref,
               widx, wts, sidx, swts, sem1, sem2, *, kp, h, w):
    b = pl.program_id(0)
    j = pl.program_id(1)
    k = b * kp + j

    pxs = [px_ref[k, i] * SCALE for i in range(8)]
    pys = [py_ref[k, i] * SCALE for i in range(8)]

    r = jax.lax.broadcasted_iota(jnp.int32, (8, 128), 0)
    l = jax.lax.broadcasted_iota(jnp.int32, (8, 128), 1)
    lin = r * 128 + l
    ph = lin // POOLED_W
    pw = lin - ph * POOLED_W
    u = pw.astype(jnp.float32) * (1.0 / POOLED_W)
    v = ph.astype(jnp.float32) * (1.0 / POOLED_H)

    s = 1.0 - u
    s2, u2 = s * s, u * u
    c0, c1, c2, c3 = s2 * s, 3.0 * u * s2, 3.0 * u2 * s, u2 * u

    def bez(p0, p1, p2, p3):
        return p0 * c0 + p1 * c1 + p2 * c2 + p3 * c3

    x0 = bez(pxs[0], pxs[1], pxs[2], pxs[3])
    x1 = bez(pxs[4], pxs[5], pxs[6], pxs[7])
    y0 = bez(pys[0], pys[1], pys[2], pys[3])
    y1 = bez(pys[4], pys[5], pys[6], pys[7])

    xc = x1 * v + x0 * (1.0 - v) - 0.5
    yc = y1 * v + y0 * (1.0 - v) - 0.5

    valid = jnp.logical_not((yc < -1.0) | (yc > h) | (xc < -1.0) | (xc > w))
    vf = valid.astype(jnp.float32)
    yq = jnp.maximum(yc, 0.0)
    xq = jnp.maximum(xc, 0.0)
    yl = jnp.minimum(jnp.floor(yq).astype(jnp.int32), h - 1)
    xl = jnp.minimum(jnp.floor(xq).astype(jnp.int32), w - 1)
    ly = jnp.where(yl >= h - 1, 0.0, yq - yl.astype(jnp.float32))
    lx = jnp.where(xl >= w - 1, 0.0, xq - xl.astype(jnp.float32))
    hy, hx = 1.0 - ly, 1.0 - lx

    widx[...] = yl * w + xl
    wts[0:8] = hy * hx * vf
    wts[8:16] = hy * lx * vf
    wts[16:24] = ly * hx * vf
    wts[24:32] = ly * lx * vf

    cp1 = pltpu.make_async_copy(widx, sidx, sem1)
    cp2 = pltpu.make_async_copy(wts, swts, sem2)
    cp1.start()
    cp2.start()
    cp1.wait()
    cp2.wait()

    def row(c, _):
        for li in range(128):
            i0 = sidx[c, li]
            slab0 = xm_ref[pl.ds(i0, 2), 0, :]
            slab1 = xm_ref[pl.ds(i0 + w, 2), 0, :]
            val = (swts[c, li] * slab0[0:1, :]
                   + swts[c + 8, li] * slab0[1:2, :]
                   + swts[c + 16, li] * slab1[0:1, :]
                   + swts[c + 24, li] * slab1[1:2, :])
            o_ref[pl.ds(c * 128 + li, 1), 0, :] = val
        return 0

    jax.lax.fori_loop(0, 8, row, 0)


def kernel(input, masks, beziers):
    n, c, h, w = input.shape
    kp = beziers.shape[1]
    hw = h * w
    hwp = hw + 2048

    xtr = input.transpose(0, 2, 3, 1).reshape(n, hw, c)
    mtr = masks.transpose(1, 2, 0).reshape(hw, c)

    chunks = hwp // 2048
    xm = pl.pallas_call(
        _prep_body,
        grid=(n, chunks),
        in_specs=[
            pl.BlockSpec((1, 2048, c), lambda b, j: (b, jnp.minimum(j, 7), 0)),
            pl.BlockSpec((2048, c), lambda b, j: (jnp.minimum(j, 7), 0)),
        ],
        out_specs=pl.BlockSpec((1, 2048, c), lambda b, j: (b, j, 0)),
        out_shape=jax.ShapeDtypeStruct((n, hwp, c), jnp.float32),
        compiler_params=pltpu.CompilerParams(
            dimension_semantics=("parallel", "arbitrary")),
    )(xtr, mtr)
    xm2 = xm.reshape(n * hwp, 1, c)

    bz = beziers.reshape(n * kp, 16)
    px = bz[:, 0::2]
    py = bz[:, 1::2]

    npts = POOLED_H * POOLED_W
    out3 = pl.pallas_call(
        functools.partial(_main_body, kp=kp, h=h, w=w),
        grid=(n, kp),
        in_specs=[
            pl.BlockSpec((hwp, 1, c), lambda b, j: (b, 0, 0)),
            pl.BlockSpec(memory_space=pltpu.SMEM),
            pl.BlockSpec(memory_space=pltpu.SMEM),
        ],
        out_specs=pl.BlockSpec((npts, 1, c), lambda b, j: (b * kp + j, 0, 0)),
        out_shape=jax.ShapeDtypeStruct((n * kp * npts, 1, c), jnp.float32),
        scratch_shapes=[
            pltpu.VMEM((8, 128), jnp.int32),
            pltpu.VMEM((32, 128), jnp.float32),
            pltpu.SMEM((8, 128), jnp.int32),
            pltpu.SMEM((32, 128), jnp.float32),
            pltpu.SemaphoreType.DMA,
            pltpu.SemaphoreType.DMA,
        ],
        compiler_params=pltpu.CompilerParams(
            dimension_semantics=("parallel", "arbitrary"),
            vmem_limit_bytes=50 * 1024 * 1024),
    )(xm2, px, py)

    return (out3.reshape(n * kp, POOLED_H, POOLED_W, c)
            .transpose(0, 3, 1, 2))


# speedup vs baseline: 14.6903x; 1.1125x over previous
import functools

import jax
import jax.numpy as jnp
from jax.experimental import pallas as pl
from jax.experimental.pallas import tpu as pltpu

POOLED_H, POOLED_W = 16, 64
SCALE = 0.25


def _prep_body(x_ref, m_ref, o_ref):
    j = pl.program_id(1)
    live = (j < 8).astype(jnp.float32)
    o_ref[0] = x_ref[0] * m_ref[...] * live


def _main_body(xm_ref, px_ref, py_ref, o_ref,
               xscr, widx, wts, sidx, swts, semx, sem1, sem2, *, kp, h, w):
    b = pl.program_id(0)
    j = pl.program_id(1)
    k = b * kp + j

    @pl.when(j == 0)
    def _():
        cpx = pltpu.make_async_copy(xm_ref.at[b], xscr, semx)
        cpx.start()
        cpx.wait()

    pxs = [px_ref[k, i] * SCALE for i in range(8)]
    pys = [py_ref[k, i] * SCALE for i in range(8)]

    r = jax.lax.broadcasted_iota(jnp.int32, (8, 128), 0)
    l = jax.lax.broadcasted_iota(jnp.int32, (8, 128), 1)
    lin = r * 128 + l
    ph = lin // POOLED_W
    pw = lin - ph * POOLED_W
    u = pw.astype(jnp.float32) * (1.0 / POOLED_W)
    v = ph.astype(jnp.float32) * (1.0 / POOLED_H)

    s = 1.0 - u
    s2, u2 = s * s, u * u
    c0, c1, c2, c3 = s2 * s, 3.0 * u * s2, 3.0 * u2 * s, u2 * u

    def bez(p0, p1, p2, p3):
        return p0 * c0 + p1 * c1 + p2 * c2 + p3 * c3

    x0 = bez(pxs[0], pxs[1], pxs[2], pxs[3])
    x1 = bez(pxs[4], pxs[5], pxs[6], pxs[7])
    y0 = bez(pys[0], pys[1], pys[2], pys[3])
    y1 = bez(pys[4], pys[5], pys[6], pys[7])

    xc = x1 * v + x0 * (1.0 - v) - 0.5
    yc = y1 * v + y0 * (1.0 - v) - 0.5

    valid = jnp.logical_not((yc < -1.0) | (yc > h) | (xc < -1.0) | (xc > w))
    vf = valid.astype(jnp.float32)
    yq = jnp.maximum(yc, 0.0)
    xq = jnp.maximum(xc, 0.0)
    yl = jnp.minimum(jnp.floor(yq).astype(jnp.int32), h - 1)
    xl = jnp.minimum(jnp.floor(xq).astype(jnp.int32), w - 1)
    ly = jnp.where(yl >= h - 1, 0.0, yq - yl.astype(jnp.float32))
    lx = jnp.where(xl >= w - 1, 0.0, xq - xl.astype(jnp.float32))
    hy, hx = 1.0 - ly, 1.0 - lx

    widx[...] = yl * w + xl
    wts[0:8] = hy * hx * vf
    wts[8:16] = hy * lx * vf
    wts[16:24] = ly * hx * vf
    wts[24:32] = ly * lx * vf

    cp1 = pltpu.make_async_copy(widx, sidx, sem1)
    cp2 = pltpu.make_async_copy(wts, swts, sem2)
    cp1.start()
    cp2.start()
    cp1.wait()
    cp2.wait()

    def row(c, _):
        for li in range(128):
            i0 = sidx[c, li]
            slab0 = xscr[pl.ds(i0, 2), 0, :]
            slab1 = xscr[pl.ds(i0 + w, 2), 0, :]
            val = (swts[c, li] * slab0[0:1, :]
                   + swts[c + 8, li] * slab0[1:2, :]
                   + swts[c + 16, li] * slab1[0:1, :]
                   + swts[c + 24, li] * slab1[1:2, :])
            o_ref[pl.ds(c * 128 + li, 1), 0, :] = val
        return 0

    jax.lax.fori_loop(0, 8, row, 0)


def kernel(input, masks, beziers):
    n, c, h, w = input.shape
    kp = beziers.shape[1]
    hw = h * w
    hwp = hw + 2048

    xtr = input.transpose(0, 2, 3, 1).reshape(n, hw, c)
    mtr = masks.transpose(1, 2, 0).reshape(hw, c)

    chunks = hwp // 2048
    xm = pl.pallas_call(
        _prep_body,
        grid=(n, chunks),
        in_specs=[
            pl.BlockSpec((1, 2048, c), lambda b, j: (b, jnp.minimum(j, 7), 0)),
            pl.BlockSpec((2048, c), lambda b, j: (jnp.minimum(j, 7), 0)),
        ],
        out_specs=pl.BlockSpec((1, 2048, c), lambda b, j: (b, j, 0)),
        out_shape=jax.ShapeDtypeStruct((n, hwp, c), jnp.float32),
        compiler_params=pltpu.CompilerParams(
            dimension_semantics=("parallel", "arbitrary")),
    )(xtr, mtr)
    xm2 = xm.reshape(n, hwp, 1, c)

    bz = beziers.reshape(n * kp, 16)
    px = bz[:, 0::2]
    py = bz[:, 1::2]

    npts = POOLED_H * POOLED_W
    out3 = pl.pallas_call(
        functools.partial(_main_body, kp=kp, h=h, w=w),
        grid=(n, kp),
        in_specs=[
            pl.BlockSpec(memory_space=pl.ANY),
            pl.BlockSpec(memory_space=pltpu.SMEM),
            pl.BlockSpec(memory_space=pltpu.SMEM),
        ],
        out_specs=pl.BlockSpec((npts, 1, c), lambda b, j: (b * kp + j, 0, 0)),
        out_shape=jax.ShapeDtypeStruct((n * kp * npts, 1, c), jnp.float32),
        scratch_shapes=[
            pltpu.VMEM((hwp, 1, c), jnp.float32),
            pltpu.VMEM((8, 128), jnp.int32),
            pltpu.VMEM((32, 128), jnp.float32),
            pltpu.SMEM((8, 128), jnp.int32),
            pltpu.SMEM((32, 128), jnp.float32),
            pltpu.SemaphoreType.DMA,
            pltpu.SemaphoreType.DMA,
            pltpu.SemaphoreType.DMA,
        ],
        compiler_params=pltpu.CompilerParams(
            dimension_semantics=("parallel", "arbitrary"),
            vmem_limit_bytes=50 * 1024 * 1024),
    )(xm2, px, py)

    return (out3.reshape(n * kp, POOLED_H, POOLED_W, c)
            .transpose(0, 3, 1, 2))
